# Initial kernel scaffold; baseline (speedup 1.0000x reference)
import jax
import jax.numpy as jnp
from jax.experimental import pallas as pl


def kernel(x, pos, edge_index, edge_attr, batch, params):
    raise NotImplementedError("write your pallas kernel here")


# speedup vs baseline: 1.0224x; 1.0224x over previous
import functools

import jax
import jax.numpy as jnp
from jax import lax
from jax.experimental import pallas as pl
from jax.experimental.pallas import tpu as pltpu
from jax.experimental.pallas import tpu_sc as plsc

N = 10000
E = 160000
D = 256
NP = 10240
EP = 163840
NC = 2
NS = 16
GCHUNK = 128
MAX_RADIUS = 10.0
N_RBF = 16

_f32 = jnp.float32


def _dot(a, b):
    return jnp.dot(a, b, preferred_element_type=_f32)


def _embed(xp, embp):
    def body(x_ref, e_ref, o_ref):
        iota = lax.broadcasted_iota(jnp.int32, (1, 128), 1)
        oh = (x_ref[...] == iota).astype(_f32)
        o_ref[...] = _dot(oh, e_ref[...])

    return pl.pallas_call(
        body,
        grid=(NP // 256,),
        in_specs=[pl.BlockSpec((256, 1), lambda i: (i, 0)),
                  pl.BlockSpec((128, 256), lambda i: (0, 0))],
        out_specs=pl.BlockSpec((256, 256), lambda i: (i, 0)),
        out_shape=jax.ShapeDtypeStruct((NP, 256), _f32),
    )(xp, embp)


def _pre(h, w1a, w1b, b1):
    def body(h_ref, wa, wb, b_ref, hr_ref, hc_ref):
        hv = h_ref[...]
        hr_ref[...] = _dot(hv, wa[...]) + b_ref[...]
        hc_ref[...] = _dot(hv, wb[...])

    return pl.pallas_call(
        body,
        grid=(NP // 256,),
        in_specs=[pl.BlockSpec((256, 256), lambda i: (i, 0)),
                  pl.BlockSpec((256, 256), lambda i: (0, 0)),
                  pl.BlockSpec((256, 256), lambda i: (0, 0)),
                  pl.BlockSpec((1, 256), lambda i: (0, 0))],
        out_specs=[pl.BlockSpec((256, 256), lambda i: (i, 0)),
                   pl.BlockSpec((256, 256), lambda i: (i, 0))],
        out_shape=[jax.ShapeDtypeStruct((NP, 256), _f32),
                   jax.ShapeDtypeStruct((NP, 256), _f32)],
    )(h, w1a, w1b, b1)


def _ew2d(gr, gc, attrp, btabp):
    def body(gr_ref, gc_ref, a_ref, bt_ref, u_ref):
        iota = lax.broadcasted_iota(jnp.int32, (1, 128), 1)
        oh = (a_ref[...] == iota).astype(_f32)
        z = gr_ref[...] + gc_ref[...] + _dot(oh, bt_ref[...])
        u_ref[...] = jnp.maximum(z, 0.0)

    return pl.pallas_call(
        body,
        grid=(EP // 640,),
        in_specs=[pl.BlockSpec((640, 256), lambda i: (i, 0)),
                  pl.BlockSpec((640, 256), lambda i: (i, 0)),
                  pl.BlockSpec((640, 1), lambda i: (i, 0)),
                  pl.BlockSpec((128, 256), lambda i: (0, 0))],
        out_specs=pl.BlockSpec((640, 256), lambda i: (i, 0)),
        out_shape=jax.ShapeDtypeStruct((EP, 256), _f32),
    )(gr, gc, attrp, btabp)


def _ew3d(gr, gc, rbfp, envp, v1c):
    def body(gr_ref, gc_ref, r_ref, e_ref, w_ref, u_ref):
        z = gr_ref[...] + gc_ref[...] + _dot(r_ref[...], w_ref[...])
        u_ref[...] = jnp.maximum(z, 0.0) * e_ref[...]

    return pl.pallas_call(
        body,
        grid=(EP // 640,),
        in_specs=[pl.BlockSpec((640, 256), lambda i: (i, 0)),
                  pl.BlockSpec((640, 256), lambda i: (i, 0)),
                  pl.BlockSpec((640, 16), lambda i: (i, 0)),
                  pl.BlockSpec((640, 1), lambda i: (i, 0)),
                  pl.BlockSpec((16, 256), lambda i: (0, 0))],
        out_specs=pl.BlockSpec((640, 256), lambda i: (i, 0)),
        out_shape=jax.ShapeDtypeStruct((EP, 256), _f32),
    )(gr, gc, rbfp, envp, v1c)


def _upd(h, s, dv, w2, b2, u1a, u1b, ub1, u2, ub2):
    def body(h_ref, s_ref, d_ref, w2r, b2r, u1ar, u1br, ub1r, u2r, ub2r, o_ref):
        agg = _dot(s_ref[...], w2r[...]) + d_ref[...] * b2r[...]
        t = jnp.maximum(_dot(h_ref[...], u1ar[...]) + _dot(agg, u1br[...])
                        + ub1r[...], 0.0)
        o_ref[...] = _dot(t, u2r[...]) + ub2r[...] + h_ref[...]

    full = pl.BlockSpec((256, 256), lambda i: (0, 0))
    bias = pl.BlockSpec((1, 256), lambda i: (0, 0))
    rows = pl.BlockSpec((256, 256), lambda i: (i, 0))
    return pl.pallas_call(
        body,
        grid=(NP // 256,),
        in_specs=[rows, rows, pl.BlockSpec((256, 1), lambda i: (i, 0)),
                  full, bias, full, full, bias, full, bias],
        out_specs=rows,
        out_shape=jax.ShapeDtypeStruct((NP, 256), _f32),
    )(h, s, dv, w2, b2, u1a, u1b, ub1, u2, ub2)


def _geom_dist(prr, pcr):
    def body(pr_ref, pc_ref, o_ref):
        d = pr_ref[...] - pc_ref[...]
        s = jnp.sum(d * d, axis=1, keepdims=True) + 1e-8
        o_ref[...] = jnp.sqrt(s)

    return pl.pallas_call(
        body,
        grid=(EP // 1024,),
        in_specs=[pl.BlockSpec((1024, 128), lambda i: (i, 0)),
                  pl.BlockSpec((1024, 128), lambda i: (i, 0))],
        out_specs=pl.BlockSpec((1024, 1), lambda i: (i, 0)),
        out_shape=jax.ShapeDtypeStruct((EP, 1), _f32),
    )(prr, pcr)


def _geom_rbf(d16, ct):
    def body(d_ref, c_ref, r_ref, e_ref):
        d = d_ref[...]
        diff = d - c_ref[...]
        r_ref[...] = jnp.exp(-(diff * diff) / (MAX_RADIUS / N_RBF))
        t = jnp.clip(d / MAX_RADIUS, 0.0, 1.0)
        e_ref[...] = 0.5 * (jnp.cos(jnp.pi * t) + 1.0)

    rows = EP * 16 // 128
    return pl.pallas_call(
        body,
        grid=(rows // 1024,),
        in_specs=[pl.BlockSpec((1024, 128), lambda i: (i, 0)),
                  pl.BlockSpec((1, 128), lambda i: (0, 0))],
        out_specs=[pl.BlockSpec((1024, 128), lambda i: (i, 0)),
                   pl.BlockSpec((1024, 128), lambda i: (i, 0))],
        out_shape=[jax.ShapeDtypeStruct((rows, 128), _f32),
                   jax.ShapeDtypeStruct((rows, 128), _f32)],
    )(d16, ct)


def _heads(h2, h3, f1a, f1b, fb1, f2, fb2, awp, abp, pwp, pbp, bap, bbp, bcp):
    def body(h2r, h3r, f1ar, f1br, fb1r, f2r, fb2r, awr, abr, pwr, pbr,
             bar, bbr, bcr, hf_ref, at_ref, pp_ref, ba_ref, bc_ref):
        t = jnp.maximum(_dot(h2r[...], f1ar[...]) + _dot(h3r[...], f1br[...])
                        + fb1r[...], 0.0)
        hf = _dot(t, f2r[...]) + fb2r[...]
        hf_ref[...] = hf
        at_ref[...] = _dot(hf, awr[...]) + abr[...]
        pp_ref[...] = _dot(hf, pwr[...]) + pbr[...]
        ba_ref[...] = _dot(hf, bar[...]) + bbr[...]
        bc_ref[...] = _dot(hf, bcr[...])

    full = pl.BlockSpec((256, 256), lambda i: (0, 0))
    w128 = pl.BlockSpec((256, 128), lambda i: (0, 0))
    bias = pl.BlockSpec((1, 256), lambda i: (0, 0))
    b128 = pl.BlockSpec((1, 128), lambda i: (0, 0))
    rows = pl.BlockSpec((256, 256), lambda i: (i, 0))
    r128 = pl.BlockSpec((256, 128), lambda i: (i, 0))
    return pl.pallas_call(
        body,
        grid=(NP // 256,),
        in_specs=[rows, rows, full, full, bias, full, bias,
                  w128, b128, w128, b128, w128, b128, w128],
        out_specs=[rows, r128, r128, r128, r128],
        out_shape=[jax.ShapeDtypeStruct((NP, 256), _f32),
                   jax.ShapeDtypeStruct((NP, 128), _f32),
                   jax.ShapeDtypeStruct((NP, 128), _f32),
                   jax.ShapeDtypeStruct((NP, 128), _f32),
                   jax.ShapeDtypeStruct((NP, 128), _f32)],
    )(h2, h3, f1a, f1b, fb1, f2, fb2, awp, abp, pwp, pbp, bap, bbp, bcp)


def _bondadd(a, b):
    def body(a_ref, b_ref, o_ref):
        o_ref[...] = a_ref[...] + b_ref[...]

    return pl.pallas_call(
        body,
        grid=(EP // 1024,),
        in_specs=[pl.BlockSpec((1024, 128), lambda i: (i, 0)),
                  pl.BlockSpec((1024, 128), lambda i: (i, 0))],
        out_specs=pl.BlockSpec((1024, 128), lambda i: (i, 0)),
        out_shape=jax.ShapeDtypeStruct((EP, 128), _f32),
    )(a, b)


def _make_dual_gather(dt):
    mesh = plsc.VectorSubcoreMesh(core_axis_name="c", subcore_axis_name="s")
    per_w = EP // (NC * NS)

    @functools.partial(
        pl.kernel,
        mesh=mesh,
        out_type=[jax.ShapeDtypeStruct((EP, dt), _f32),
                  jax.ShapeDtypeStruct((EP, dt), _f32)],
        scratch_types=[
            pltpu.VMEM((GCHUNK,), jnp.int32),
            pltpu.VMEM((GCHUNK,), jnp.int32),
            pltpu.VMEM((GCHUNK, dt), _f32),
            pltpu.VMEM((GCHUNK, dt), _f32),
            pltpu.SemaphoreType.DMA,
            pltpu.SemaphoreType.DMA,
        ],
    )
    def k(ta, tb, ia, ib, oa, ob, ia_v, ib_v, va, vb, sema, semb):
        wid = lax.axis_index("s") * NC + lax.axis_index("c")
        base0 = wid * per_w

        def body(i, carry):
            base = base0 + i * GCHUNK
            pltpu.sync_copy(ia.at[pl.ds(base, GCHUNK)], ia_v)
            pltpu.sync_copy(ib.at[pl.ds(base, GCHUNK)], ib_v)
            ca = pltpu.async_copy(ta.at[ia_v], va, sema)
            cb = pltpu.async_copy(tb.at[ib_v], vb, semb)
            ca.wait()
            cb.wait()
            pltpu.sync_copy(va, oa.at[pl.ds(base, GCHUNK)])
            pltpu.sync_copy(vb, ob.at[pl.ds(base, GCHUNK)])
            return carry

        lax.fori_loop(0, per_w // GCHUNK, body, 0)

    return k


def _scatter_add(u, colp, zeros128):
    mesh = plsc.VectorSubcoreMesh(core_axis_name="c", subcore_axis_name="s")
    per_s = EP // NS
    rows = NP // NS

    @functools.partial(
        pl.kernel,
        mesh=mesh,
        out_type=jax.ShapeDtypeStruct((NP, 256), _f32),
        scratch_types=[
            pltpu.VMEM((GCHUNK,), jnp.int32),
            pltpu.VMEM((GCHUNK, 128), _f32),
            pltpu.VMEM_SHARED((NP, 128), _f32),
        ],
    )
    def k(u_hbm, col_hbm, z_hbm, out, idx_v, u_v, acc):
        c = lax.axis_index("c")
        s = lax.axis_index("s")
        pltpu.sync_copy(z_hbm.at[pl.ds(s * rows, rows)],
                        acc.at[pl.ds(s * rows, rows)])
        plsc.subcore_barrier()

        def body(i, carry):
            base = s * per_s + i * GCHUNK
            pltpu.sync_copy(col_hbm.at[pl.ds(base, GCHUNK)], idx_v)
            pltpu.sync_copy(u_hbm.at[pl.ds(base, GCHUNK), pl.ds(c * 128, 128)],
                            u_v)
            pltpu.sync_copy(u_v, acc.at[idx_v], add=True)
            return carry

        lax.fori_loop(0, per_s // GCHUNK, body, 0)
        plsc.subcore_barrier()
        pltpu.sync_copy(acc.at[pl.ds(s * rows, rows)],
                        out.at[pl.ds(s * rows, rows), pl.ds(c * 128, 128)])

    return k(u, colp, zeros128)


def _deg_scatter(u0, colp, zeros128):
    mesh = plsc.VectorSubcoreMesh(core_axis_name="c", subcore_axis_name="s")
    per_s = EP // NS
    rows = NP // NS

    @functools.partial(
        pl.kernel,
        mesh=mesh,
        out_type=jax.ShapeDtypeStruct((NP, 128), _f32),
        scratch_types=[
            pltpu.VMEM((GCHUNK,), jnp.int32),
            pltpu.VMEM((GCHUNK, 128), _f32),
            pltpu.VMEM_SHARED((NP, 128), _f32),
        ],
    )
    def k(u_hbm, col_hbm, z_hbm, out, idx_v, u_v, acc):
        c = lax.axis_index("c")
        s = lax.axis_index("s")

        @pl.when(c == 0)
        def _():
            pltpu.sync_copy(z_hbm.at[pl.ds(s * rows, rows)],
                            acc.at[pl.ds(s * rows, rows)])
            plsc.subcore_barrier()

            def body(i, carry):
                base = s * per_s + i * GCHUNK
                pltpu.sync_copy(col_hbm.at[pl.ds(base, GCHUNK)], idx_v)
                pltpu.sync_copy(u_hbm.at[pl.ds(base, GCHUNK)], u_v)
                pltpu.sync_copy(u_v, acc.at[idx_v], add=True)
                return carry

            lax.fori_loop(0, per_s // GCHUNK, body, 0)
            plsc.subcore_barrier()
            pltpu.sync_copy(acc.at[pl.ds(s * rows, rows)],
                            out.at[pl.ds(s * rows, rows)])

    return k(u0, colp, zeros128)


def kernel(x, pos, edge_index, edge_attr, batch, params):
    gather256 = _make_dual_gather(256)
    gather128 = _make_dual_gather(128)

    row = edge_index[0].astype(jnp.int32)
    col = edge_index[1].astype(jnp.int32)
    pad_e = EP - E
    rowp = jnp.concatenate([row, jnp.full((pad_e,), N, jnp.int32)])
    colp = jnp.concatenate([col, jnp.full((pad_e,), N, jnp.int32)])
    attrp = jnp.pad(edge_attr.astype(jnp.int32), ((0, pad_e), (0, 0)))
    xp = jnp.pad(x.astype(jnp.int32), ((0, NP - N), (0, 0)))
    posp = jnp.pad(pos, ((0, NP - N), (0, 125)))
    embp = jnp.pad(params["atom_emb"], ((0, 28), (0, 0)))

    zeros128 = jnp.zeros((NP, 128), _f32)

    h0 = _embed(xp, embp)

    pr, pc = gather128(posp, posp, rowp, colp)
    dist_e = _geom_dist(pr, pc)
    d16 = jnp.broadcast_to(dist_e, (EP, 16)).reshape(EP * 16 // 128, 128)
    centers = jnp.linspace(0.0, MAX_RADIUS, N_RBF)
    ct = jnp.tile(centers, 8).reshape(1, 128)
    rbf_r, env_r = _geom_rbf(d16, ct)
    rbfp = rbf_r.reshape(EP, 16)
    envp = env_r.reshape(EP, 16)[:, :1]

    u0 = jnp.concatenate([jnp.ones((EP, 1), _f32), envp,
                          jnp.zeros((EP, 126), _f32)], axis=1)
    degenv = _deg_scatter(u0, colp, zeros128)
    deg = degenv[:, :1]
    envsum = degenv[:, 1:2]

    h2 = h0
    for lp in params["g2d"]:
        w1, b1 = lp["msg"]["w1"], lp["msg"]["b1"]
        w2, b2 = lp["msg"]["w2"], lp["msg"]["b2"]
        hr, hc = _pre(h2, w1[:D], w1[D:2 * D], b1.reshape(1, D))
        gr, gc = gather256(hr, hc, rowp, colp)
        btabp = jnp.pad(params["bond_emb"] @ w1[2 * D:], ((0, 123), (0, 0)))
        u = _ew2d(gr, gc, attrp, btabp)
        s = _scatter_add(u, colp, zeros128)
        up = lp["upd"]
        h2 = _upd(h2, s, deg, w2, b2.reshape(1, D),
                  up["w1"][:D], up["w1"][D:], up["b1"].reshape(1, D),
                  up["w2"], up["b2"].reshape(1, D))

    h3 = h0
    for lp in params["e3"]:
        w1, b1 = lp["msg"]["w1"], lp["msg"]["b1"]
        w2, b2 = lp["msg"]["w2"], lp["msg"]["b2"]
        hr, hc = _pre(h3, w1[:D], w1[D:2 * D], b1.reshape(1, D))
        gr, gc = gather256(hr, hc, rowp, colp)
        u = _ew3d(gr, gc, rbfp, envp, w1[2 * D:])
        s = _scatter_add(u, colp, zeros128)
        up = lp["upd"]
        h3 = _upd(h3, s, envsum, w2, b2.reshape(1, D),
                  up["w1"][:D], up["w1"][D:], up["b1"].reshape(1, D),
                  up["w2"], up["b2"].reshape(1, D))

    fu = params["fuse"]
    aw, ab = params["atom_head"]
    pw, pb = params["pos_head"]
    bw, bb = params["bond_head"]
    awp = jnp.pad(aw, ((0, 0), (0, 28)))
    abp = jnp.pad(ab.reshape(1, -1), ((0, 0), (0, 28)))
    pwp = jnp.pad(pw, ((0, 0), (0, 125)))
    pbp = jnp.pad(pb.reshape(1, -1), ((0, 0), (0, 125)))
    bap = jnp.pad(bw[:D], ((0, 0), (0, 123)))
    bbp = jnp.pad(bb.reshape(1, -1), ((0, 0), (0, 123)))
    bcp = jnp.pad(bw[D:], ((0, 0), (0, 123)))
    hf, atomp, posp, batab, bctab = _heads(
        h2, h3, fu["w1"][:D], fu["w1"][D:], fu["b1"].reshape(1, D),
        fu["w2"], fu["b2"].reshape(1, D), awp, abp, pwp, pbp, bap, bbp, bcp)

    gba, gbc = gather128(batab, bctab, rowp, colp)
    bond_r = _bondadd(gba, gbc)
    bond_logits = bond_r[:E, :5]

    return (atomp[:N, :100], posp[:N, :3], bond_logits, hf[:N])


# speedup vs baseline: 1.0234x; 1.0009x over previous
import functools

import jax
import jax.numpy as jnp
from jax import lax
from jax.experimental import pallas as pl
from jax.experimental.pallas import tpu as pltpu
from jax.experimental.pallas import tpu_sc as plsc

N = 10000
E = 160000
D = 256
NP = 10240
EP = 163840
NC = 2
NS = 16
GCHUNK = 128
MAX_RADIUS = 10.0
N_RBF = 16

_f32 = jnp.float32


def _dot(a, b):
    return jnp.dot(a, b, preferred_element_type=_f32)


def _embed(xp, embp):
    def body(x_ref, e_ref, o_ref):
        iota = lax.broadcasted_iota(jnp.int32, (1, 128), 1)
        oh = (x_ref[...] == iota).astype(_f32)
        o_ref[...] = _dot(oh, e_ref[...])

    return pl.pallas_call(
        body,
        grid=(NP // 256,),
        in_specs=[pl.BlockSpec((256, 1), lambda i: (i, 0)),
                  pl.BlockSpec((128, 256), lambda i: (0, 0))],
        out_specs=pl.BlockSpec((256, 256), lambda i: (i, 0)),
        out_shape=jax.ShapeDtypeStruct((NP, 256), _f32),
    )(xp, embp)


def _pre(h, w1a, w1b, b1):
    def body(h_ref, wa, wb, b_ref, hr_ref, hc_ref):
        hv = h_ref[...]
        hr_ref[...] = _dot(hv, wa[...]) + b_ref[...]
        hc_ref[...] = _dot(hv, wb[...])

    return pl.pallas_call(
        body,
        grid=(NP // 256,),
        in_specs=[pl.BlockSpec((256, 256), lambda i: (i, 0)),
                  pl.BlockSpec((256, 256), lambda i: (0, 0)),
                  pl.BlockSpec((256, 256), lambda i: (0, 0)),
                  pl.BlockSpec((1, 256), lambda i: (0, 0))],
        out_specs=[pl.BlockSpec((256, 256), lambda i: (i, 0)),
                   pl.BlockSpec((256, 256), lambda i: (i, 0))],
        out_shape=[jax.ShapeDtypeStruct((NP, 256), _f32),
                   jax.ShapeDtypeStruct((NP, 256), _f32)],
    )(h, w1a, w1b, b1)


def _ew2d(gr, gc, attrp, btabp):
    def body(gr_ref, gc_ref, a_ref, bt_ref, u_ref):
        iota = lax.broadcasted_iota(jnp.int32, (1, 128), 1)
        oh = (a_ref[...] == iota).astype(_f32)
        z = (gr_ref[...].astype(_f32) + gc_ref[...].astype(_f32)
             + _dot(oh, bt_ref[...]))
        u_ref[...] = jnp.maximum(z, 0.0)

    return pl.pallas_call(
        body,
        grid=(EP // 640,),
        in_specs=[pl.BlockSpec((640, 256), lambda i: (i, 0)),
                  pl.BlockSpec((640, 256), lambda i: (i, 0)),
                  pl.BlockSpec((640, 1), lambda i: (i, 0)),
                  pl.BlockSpec((128, 256), lambda i: (0, 0))],
        out_specs=pl.BlockSpec((640, 256), lambda i: (i, 0)),
        out_shape=jax.ShapeDtypeStruct((EP, 256), _f32),
    )(gr, gc, attrp, btabp)


def _ew3d(gr, gc, rbfp, envp, v1c):
    def body(gr_ref, gc_ref, r_ref, e_ref, w_ref, u_ref):
        z = (gr_ref[...].astype(_f32) + gc_ref[...].astype(_f32)
             + _dot(r_ref[...], w_ref[...]))
        u_ref[...] = jnp.maximum(z, 0.0) * e_ref[...]

    return pl.pallas_call(
        body,
        grid=(EP // 640,),
        in_specs=[pl.BlockSpec((640, 256), lambda i: (i, 0)),
                  pl.BlockSpec((640, 256), lambda i: (i, 0)),
                  pl.BlockSpec((640, 16), lambda i: (i, 0)),
                  pl.BlockSpec((640, 1), lambda i: (i, 0)),
                  pl.BlockSpec((16, 256), lambda i: (0, 0))],
        out_specs=pl.BlockSpec((640, 256), lambda i: (i, 0)),
        out_shape=jax.ShapeDtypeStruct((EP, 256), _f32),
    )(gr, gc, rbfp, envp, v1c)


def _upd(h, s, dv, w2, b2, u1a, u1b, ub1, u2, ub2):
    def body(h_ref, s_ref, d_ref, w2r, b2r, u1ar, u1br, ub1r, u2r, ub2r, o_ref):
        agg = _dot(s_ref[...], w2r[...]) + d_ref[...] * b2r[...]
        t = jnp.maximum(_dot(h_ref[...], u1ar[...]) + _dot(agg, u1br[...])
                        + ub1r[...], 0.0)
        o_ref[...] = _dot(t, u2r[...]) + ub2r[...] + h_ref[...]

    full = pl.BlockSpec((256, 256), lambda i: (0, 0))
    bias = pl.BlockSpec((1, 256), lambda i: (0, 0))
    rows = pl.BlockSpec((256, 256), lambda i: (i, 0))
    return pl.pallas_call(
        body,
        grid=(NP // 256,),
        in_specs=[rows, rows, pl.BlockSpec((256, 1), lambda i: (i, 0)),
                  full, bias, full, full, bias, full, bias],
        out_specs=rows,
        out_shape=jax.ShapeDtypeStruct((NP, 256), _f32),
    )(h, s, dv, w2, b2, u1a, u1b, ub1, u2, ub2)


def _geom_dist(prr, pcr):
    def body(pr_ref, pc_ref, o_ref):
        d = pr_ref[...] - pc_ref[...]
        s = jnp.sum(d * d, axis=1, keepdims=True) + 1e-8
        o_ref[...] = jnp.sqrt(s)

    return pl.pallas_call(
        body,
        grid=(EP // 1024,),
        in_specs=[pl.BlockSpec((1024, 128), lambda i: (i, 0)),
                  pl.BlockSpec((1024, 128), lambda i: (i, 0))],
        out_specs=pl.BlockSpec((1024, 1), lambda i: (i, 0)),
        out_shape=jax.ShapeDtypeStruct((EP, 1), _f32),
    )(prr, pcr)


def _geom_rbf(d16, ct):
    def body(d_ref, c_ref, r_ref, e_ref):
        d = d_ref[...]
        diff = d - c_ref[...]
        r_ref[...] = jnp.exp(-(diff * diff) / (MAX_RADIUS / N_RBF))
        t = jnp.clip(d / MAX_RADIUS, 0.0, 1.0)
        e_ref[...] = 0.5 * (jnp.cos(jnp.pi * t) + 1.0)

    rows = EP * 16 // 128
    return pl.pallas_call(
        body,
        grid=(rows // 1024,),
        in_specs=[pl.BlockSpec((1024, 128), lambda i: (i, 0)),
                  pl.BlockSpec((1, 128), lambda i: (0, 0))],
        out_specs=[pl.BlockSpec((1024, 128), lambda i: (i, 0)),
                   pl.BlockSpec((1024, 128), lambda i: (i, 0))],
        out_shape=[jax.ShapeDtypeStruct((rows, 128), _f32),
                   jax.ShapeDtypeStruct((rows, 128), _f32)],
    )(d16, ct)


def _heads(h2, h3, f1a, f1b, fb1, f2, fb2, awp, abp, pwp, pbp, bap, bbp, bcp):
    def body(h2r, h3r, f1ar, f1br, fb1r, f2r, fb2r, awr, abr, pwr, pbr,
             bar, bbr, bcr, hf_ref, at_ref, pp_ref, ba_ref, bc_ref):
        t = jnp.maximum(_dot(h2r[...], f1ar[...]) + _dot(h3r[...], f1br[...])
                        + fb1r[...], 0.0)
        hf = _dot(t, f2r[...]) + fb2r[...]
        hf_ref[...] = hf
        at_ref[...] = _dot(hf, awr[...]) + abr[...]
        pp_ref[...] = _dot(hf, pwr[...]) + pbr[...]
        ba_ref[...] = _dot(hf, bar[...]) + bbr[...]
        bc_ref[...] = _dot(hf, bcr[...])

    full = pl.BlockSpec((256, 256), lambda i: (0, 0))
    w128 = pl.BlockSpec((256, 128), lambda i: (0, 0))
    bias = pl.BlockSpec((1, 256), lambda i: (0, 0))
    b128 = pl.BlockSpec((1, 128), lambda i: (0, 0))
    rows = pl.BlockSpec((256, 256), lambda i: (i, 0))
    r128 = pl.BlockSpec((256, 128), lambda i: (i, 0))
    return pl.pallas_call(
        body,
        grid=(NP // 256,),
        in_specs=[rows, rows, full, full, bias, full, bias,
                  w128, b128, w128, b128, w128, b128, w128],
        out_specs=[rows, r128, r128, r128, r128],
        out_shape=[jax.ShapeDtypeStruct((NP, 256), _f32),
                   jax.ShapeDtypeStruct((NP, 128), _f32),
                   jax.ShapeDtypeStruct((NP, 128), _f32),
                   jax.ShapeDtypeStruct((NP, 128), _f32),
                   jax.ShapeDtypeStruct((NP, 128), _f32)],
    )(h2, h3, f1a, f1b, fb1, f2, fb2, awp, abp, pwp, pbp, bap, bbp, bcp)


def _bondadd(a, b):
    def body(a_ref, b_ref, o_ref):
        o_ref[...] = a_ref[...] + b_ref[...]

    return pl.pallas_call(
        body,
        grid=(EP // 1024,),
        in_specs=[pl.BlockSpec((1024, 128), lambda i: (i, 0)),
                  pl.BlockSpec((1024, 128), lambda i: (i, 0))],
        out_specs=pl.BlockSpec((1024, 128), lambda i: (i, 0)),
        out_shape=jax.ShapeDtypeStruct((EP, 128), _f32),
    )(a, b)


def _make_dual_gather(dt, dtype):
    mesh = plsc.VectorSubcoreMesh(core_axis_name="c", subcore_axis_name="s")
    per_w = EP // (NC * NS)

    @functools.partial(
        pl.kernel,
        mesh=mesh,
        out_type=[jax.ShapeDtypeStruct((EP, dt), dtype),
                  jax.ShapeDtypeStruct((EP, dt), dtype)],
        scratch_types=[
            pltpu.VMEM((GCHUNK,), jnp.int32),
            pltpu.VMEM((GCHUNK,), jnp.int32),
            pltpu.VMEM((GCHUNK, dt), dtype),
            pltpu.VMEM((GCHUNK, dt), dtype),
            pltpu.SemaphoreType.DMA,
            pltpu.SemaphoreType.DMA,
        ],
    )
    def k(ta, tb, ia, ib, oa, ob, ia_v, ib_v, va, vb, sema, semb):
        wid = lax.axis_index("s") * NC + lax.axis_index("c")
        base0 = wid * per_w

        def body(i, carry):
            base = base0 + i * GCHUNK
            pltpu.sync_copy(ia.at[pl.ds(base, GCHUNK)], ia_v)
            pltpu.sync_copy(ib.at[pl.ds(base, GCHUNK)], ib_v)
            ca = pltpu.async_copy(ta.at[ia_v], va, sema)
            cb = pltpu.async_copy(tb.at[ib_v], vb, semb)
            ca.wait()
            cb.wait()
            pltpu.sync_copy(va, oa.at[pl.ds(base, GCHUNK)])
            pltpu.sync_copy(vb, ob.at[pl.ds(base, GCHUNK)])
            return carry

        lax.fori_loop(0, per_w // GCHUNK, body, 0)

    return k


def _scatter_add(u, colp, zeros128):
    mesh = plsc.VectorSubcoreMesh(core_axis_name="c", subcore_axis_name="s")
    per_s = EP // NS
    rows = NP // NS

    @functools.partial(
        pl.kernel,
        mesh=mesh,
        out_type=jax.ShapeDtypeStruct((NP, 256), _f32),
        scratch_types=[
            pltpu.VMEM((GCHUNK,), jnp.int32),
            pltpu.VMEM((GCHUNK, 128), _f32),
            pltpu.VMEM_SHARED((NP, 128), _f32),
        ],
    )
    def k(u_hbm, col_hbm, z_hbm, out, idx_v, u_v, acc):
        c = lax.axis_index("c")
        s = lax.axis_index("s")
        pltpu.sync_copy(z_hbm.at[pl.ds(s * rows, rows)],
                        acc.at[pl.ds(s * rows, rows)])
        plsc.subcore_barrier()

        def body(i, carry):
            base = s * per_s + i * GCHUNK
            pltpu.sync_copy(col_hbm.at[pl.ds(base, GCHUNK)], idx_v)
            pltpu.sync_copy(u_hbm.at[pl.ds(base, GCHUNK), pl.ds(c * 128, 128)],
                            u_v)
            pltpu.sync_copy(u_v, acc.at[idx_v], add=True)
            return carry

        lax.fori_loop(0, per_s // GCHUNK, body, 0)
        plsc.subcore_barrier()
        pltpu.sync_copy(acc.at[pl.ds(s * rows, rows)],
                        out.at[pl.ds(s * rows, rows), pl.ds(c * 128, 128)])

    return k(u, colp, zeros128)


def _deg_scatter(u0, colp, zeros128):
    mesh = plsc.VectorSubcoreMesh(core_axis_name="c", subcore_axis_name="s")
    per_s = EP // NS
    rows = NP // NS

    @functools.partial(
        pl.kernel,
        mesh=mesh,
        out_type=jax.ShapeDtypeStruct((NP, 128), _f32),
        scratch_types=[
            pltpu.VMEM((GCHUNK,), jnp.int32),
            pltpu.VMEM((GCHUNK, 128), _f32),
            pltpu.VMEM_SHARED((NP, 128), _f32),
        ],
    )
    def k(u_hbm, col_hbm, z_hbm, out, idx_v, u_v, acc):
        c = lax.axis_index("c")
        s = lax.axis_index("s")

        @pl.when(c == 0)
        def _():
            pltpu.sync_copy(z_hbm.at[pl.ds(s * rows, rows)],
                            acc.at[pl.ds(s * rows, rows)])
            plsc.subcore_barrier()

            def body(i, carry):
                base = s * per_s + i * GCHUNK
                pltpu.sync_copy(col_hbm.at[pl.ds(base, GCHUNK)], idx_v)
                pltpu.sync_copy(u_hbm.at[pl.ds(base, GCHUNK)], u_v)
                pltpu.sync_copy(u_v, acc.at[idx_v], add=True)
                return carry

            lax.fori_loop(0, per_s // GCHUNK, body, 0)
            plsc.subcore_barrier()
            pltpu.sync_copy(acc.at[pl.ds(s * rows, rows)],
                            out.at[pl.ds(s * rows, rows)])

    return k(u0, colp, zeros128)


def kernel(x, pos, edge_index, edge_attr, batch, params):
    gather256 = _make_dual_gather(256, _f32)
    gather128 = _make_dual_gather(128, _f32)

    row = edge_index[0].astype(jnp.int32)
    col = edge_index[1].astype(jnp.int32)
    pad_e = EP - E
    rowp = jnp.concatenate([row, jnp.full((pad_e,), N, jnp.int32)])
    colp = jnp.concatenate([col, jnp.full((pad_e,), N, jnp.int32)])
    attrp = jnp.pad(edge_attr.astype(jnp.int32), ((0, pad_e), (0, 0)))
    xp = jnp.pad(x.astype(jnp.int32), ((0, NP - N), (0, 0)))
    posp = jnp.pad(pos, ((0, NP - N), (0, 125)))
    embp = jnp.pad(params["atom_emb"], ((0, 28), (0, 0)))

    zeros128 = jnp.zeros((NP, 128), _f32)

    h0 = _embed(xp, embp)

    pr, pc = gather128(posp, posp, rowp, colp)
    dist_e = _geom_dist(pr, pc)
    d16 = jnp.broadcast_to(dist_e, (EP, 16)).reshape(EP * 16 // 128, 128)
    centers = jnp.linspace(0.0, MAX_RADIUS, N_RBF)
    ct = jnp.tile(centers, 8).reshape(1, 128)
    rbf_r, env_r = _geom_rbf(d16, ct)
    rbfp = rbf_r.reshape(EP, 16)
    envp = env_r.reshape(EP, 16)[:, :1]

    u0 = jnp.concatenate([jnp.ones((EP, 1), _f32), envp,
                          jnp.zeros((EP, 126), _f32)], axis=1)
    degenv = _deg_scatter(u0, colp, zeros128)
    deg = degenv[:, :1]
    envsum = degenv[:, 1:2]

    h2 = h0
    for lp in params["g2d"]:
        w1, b1 = lp["msg"]["w1"], lp["msg"]["b1"]
        w2, b2 = lp["msg"]["w2"], lp["msg"]["b2"]
        hr, hc = _pre(h2, w1[:D], w1[D:2 * D], b1.reshape(1, D))
        gr, gc = gather256(hr, hc, rowp, colp)
        btabp = jnp.pad(params["bond_emb"] @ w1[2 * D:], ((0, 123), (0, 0)))
        u = _ew2d(gr, gc, attrp, btabp)
        s = _scatter_add(u, colp, zeros128)
        up = lp["upd"]
        h2 = _upd(h2, s, deg, w2, b2.reshape(1, D),
                  up["w1"][:D], up["w1"][D:], up["b1"].reshape(1, D),
                  up["w2"], up["b2"].reshape(1, D))

    h3 = h0
    for lp in params["e3"]:
        w1, b1 = lp["msg"]["w1"], lp["msg"]["b1"]
        w2, b2 = lp["msg"]["w2"], lp["msg"]["b2"]
        hr, hc = _pre(h3, w1[:D], w1[D:2 * D], b1.reshape(1, D))
        gr, gc = gather256(hr, hc, rowp, colp)
        u = _ew3d(gr, gc, rbfp, envp, w1[2 * D:])
        s = _scatter_add(u, colp, zeros128)
        up = lp["upd"]
        h3 = _upd(h3, s, envsum, w2, b2.reshape(1, D),
                  up["w1"][:D], up["w1"][D:], up["b1"].reshape(1, D),
                  up["w2"], up["b2"].reshape(1, D))

    fu = params["fuse"]
    aw, ab = params["atom_head"]
    pw, pb = params["pos_head"]
    bw, bb = params["bond_head"]
    awp = jnp.pad(aw, ((0, 0), (0, 28)))
    abp = jnp.pad(ab.reshape(1, -1), ((0, 0), (0, 28)))
    pwp = jnp.pad(pw, ((0, 0), (0, 125)))
    pbp = jnp.pad(pb.reshape(1, -1), ((0, 0), (0, 125)))
    bap = jnp.pad(bw[:D], ((0, 0), (0, 123)))
    bbp = jnp.pad(bb.reshape(1, -1), ((0, 0), (0, 123)))
    bcp = jnp.pad(bw[D:], ((0, 0), (0, 123)))
    hf, atomp, posp, batab, bctab = _heads(
        h2, h3, fu["w1"][:D], fu["w1"][D:], fu["b1"].reshape(1, D),
        fu["w2"], fu["b2"].reshape(1, D), awp, abp, pwp, pbp, bap, bbp, bcp)

    gba, gbc = gather128(batab, bctab, rowp, colp)
    bond_r = _bondadd(gba, gbc)
    bond_logits = bond_r[:E, :5]

    return (atomp[:N, :100], posp[:N, :3], bond_logits, hf[:N])
